# baseline (speedup 1.0000x reference)
import functools
import math

import jax
import jax.numpy as jnp
from jax import lax
from jax.experimental import pallas as pl
from jax.experimental.pallas import tpu as pltpu
from jax.experimental.pallas import tpu_sc as plsc

NC = 2
NS = 16
NW = NC * NS
LANES = 16

SUB = 128
K = 2
TCH = 512


def _make_transpose_kernel(V, D):
    pack = SUB // D
    nch = V // TCH
    orows = TCH // pack
    niter = (nch + NW - 1) // NW
    niter2 = (niter + 1) // 2

    mesh = plsc.VectorSubcoreMesh(core_axis_name="c", subcore_axis_name="s")

    @functools.partial(
        pl.kernel,
        out_type=jax.ShapeDtypeStruct((V * D // SUB, SUB), jnp.float32),
        mesh=mesh,
        scratch_types=[
            pltpu.VMEM((2, D, TCH), jnp.float32),
            pltpu.VMEM((2, orows, SUB), jnp.float32),
            pltpu.SemaphoreType.DMA,
            pltpu.SemaphoreType.DMA,
            pltpu.SemaphoreType.DMA,
        ],
        compiler_params=pltpu.CompilerParams(needs_layout_passes=False),
    )
    def trans(tbl_t_hbm, tail_hbm, out_hbm, in_v, out_v, sem_i, so0, so1):
        wid = lax.axis_index("s") * NC + lax.axis_index("c")
        iota = lax.iota(jnp.int32, LANES)
        orow_off = iota >> 2
        ocol_base = (iota & (pack - 1)) * D

        def do_chunk(in_ref, out_ref):
            for d0 in range(D):
                diag = (iota + d0) & (D - 1)
                ocol = ocol_base + diag

                def gbody(g, c):
                    vcol = iota + g * LANES
                    orow = orow_off + g * (LANES // pack)
                    val = plsc.load_gather(in_ref, [diag, vcol])
                    plsc.store_scatter(out_ref, [orow, ocol], val)
                    return c

                lax.fori_loop(0, TCH // LANES, gbody, 0)

        def issue(s, buf):
            ch = wid + s * NW

            @pl.when(ch < nch)
            def _():
                v0 = pl.multiple_of(ch * TCH, SUB)
                pltpu.async_copy(
                    tbl_t_hbm.at[:, pl.ds(v0, TCH)], in_v.at[buf], sem_i
                )

        def drain_i(s, buf):
            ch = wid + s * NW

            @pl.when(ch < nch)
            def _():
                v0 = pl.multiple_of(ch * TCH, SUB)
                pltpu.make_async_copy(
                    tbl_t_hbm.at[:, pl.ds(v0, TCH)], in_v.at[buf], sem_i
                ).wait()

        def proc(s, buf, p, sem_o):
            ch = wid + s * NW

            @pl.when(ch < nch)
            def _():
                @pl.when(p > 0)
                def _():
                    pltpu.make_async_copy(
                        out_v.at[buf], out_hbm.at[pl.ds(0, orows)], sem_o
                    ).wait()

                do_chunk(in_v.at[buf], out_v.at[buf])
                o0 = pl.multiple_of(ch * orows, orows)
                pltpu.async_copy(out_v.at[buf], out_hbm.at[pl.ds(o0, orows)],
                                 sem_o)

        issue(0, 0)

        def pair_body(p, carry):
            s0 = 2 * p
            issue(s0 + 1, 1)
            drain_i(s0, 0)
            proc(s0, 0, p, so0)
            issue(s0 + 2, 0)
            drain_i(s0 + 1, 1)
            proc(s0 + 1, 1, p, so1)
            return carry

        lax.fori_loop(0, niter2, pair_body, 0)

        assert nch >= 3 * NW
        pltpu.make_async_copy(
            out_v.at[0], out_hbm.at[pl.ds(0, orows)], so0
        ).wait()
        pltpu.make_async_copy(
            out_v.at[1], out_hbm.at[pl.ds(0, orows)], so1
        ).wait()

        rem = V - nch * TCH
        a = (rem // SUB) * SUB
        if a:
            @pl.when(wid == 0)
            def _():
                pltpu.sync_copy(
                    tbl_t_hbm.at[:, pl.ds(nch * TCH, a)],
                    in_v.at[0, :, pl.ds(0, a)],
                )
                def gbody(g, c):
                    vcol = iota + g * LANES
                    orow = orow_off + g * (LANES // pack)
                    for d0 in range(D):
                        diag = (iota + d0) & (D - 1)
                        val = plsc.load_gather(in_v.at[0], [diag, vcol])
                        plsc.store_scatter(out_v.at[0],
                                           [orow, ocol_base + diag], val)
                    return c

                lax.fori_loop(0, a // LANES, gbody, 0)
                pltpu.sync_copy(
                    out_v.at[0, pl.ds(0, a // pack)],
                    out_hbm.at[pl.ds((nch * TCH) // pack, a // pack)],
                )

        t16 = (V % SUB) * D // SUB
        if t16:
            @pl.when(wid == 1)
            def _():
                pltpu.sync_copy(tail_hbm, out_v.at[0, pl.ds(0, t16)])
                pltpu.sync_copy(
                    out_v.at[0, pl.ds(0, t16)],
                    out_hbm.at[pl.ds((V * D // SUB) - t16, t16)],
                )

    return trans


def _make_gather_kernel(L, B, D, scale):
    cpl = B // (K * SUB)
    nblk = L * cpl // NW
    assert nblk % 2 == 0
    pack = SUB // D
    BL = K * SUB

    mesh = plsc.VectorSubcoreMesh(core_axis_name="c", subcore_axis_name="s")

    @functools.partial(
        pl.kernel,
        out_type=jax.ShapeDtypeStruct((L, D, B), jnp.float32),
        mesh=mesh,
        scratch_types=[
            pltpu.VMEM((2, BL), jnp.int32),
            pltpu.VMEM((2, K, SUB), jnp.int32),
            pltpu.VMEM((2, BL // LANES, LANES), jnp.int32),
            pltpu.VMEM((2, BL, SUB), jnp.float32),
            pltpu.VMEM((2, D, BL), jnp.float32),
            pltpu.SemaphoreType.DMA,
            pltpu.SemaphoreType.DMA,
            pltpu.SemaphoreType.DMA,
        ],
        compiler_params=pltpu.CompilerParams(needs_layout_passes=False),
    )
    def emb(tokens_hbm, table_hbm, out_hbm, idv_v, idq_v, rem_v, rows_v,
            out_v, sem_g, sem_o0, sem_o1):
        wid = lax.axis_index("s") * NC + lax.axis_index("c")
        iota = lax.iota(jnp.int32, LANES)
        base = wid * nblk

        def lb0(bid):
            return bid // cpl, (bid % cpl) * BL

        def issue(bid, buf):
            l, b0 = lb0(bid)
            pltpu.sync_copy(tokens_hbm.at[l, pl.ds(b0, BL)], idv_v.at[buf])
            for j in range(K):
                for g in range(SUB // LANES):
                    iv = idv_v[buf, pl.ds(j * SUB + g * LANES, LANES)]
                    rem_v[buf, j * (SUB // LANES) + g, :] = iv & (pack - 1)
                    idq_v[buf, j, pl.ds(g * LANES, LANES)] = iv >> 2
            for j in range(K):
                pltpu.async_copy(
                    table_hbm.at[idq_v.at[buf, j]],
                    rows_v.at[buf, pl.ds(j * SUB, SUB)],
                    sem_g,
                )

        def drain_g(buf):
            for j in range(K):
                pltpu.make_async_copy(
                    table_hbm.at[idq_v.at[buf, j]],
                    rows_v.at[buf, pl.ds(j * SUB, SUB)],
                    sem_g,
                ).wait()

        def proc(bid, buf, p, sem_o):
            l, b0 = lb0(bid)

            @pl.when(p > 0)
            def _():
                l2, b02 = lb0(bid - 2)
                pltpu.make_async_copy(
                    out_v.at[buf], out_hbm.at[l2, :, pl.ds(b02, BL)], sem_o
                ).wait()

            def sel_body(jg, c2):
                rv = rem_v[buf, jg, :]
                colbase = rv * D
                rowvec = iota + jg * LANES
                for d0 in range(D):
                    diag = (iota + d0) & (D - 1)
                    val = plsc.load_gather(
                        rows_v.at[buf], [rowvec, colbase + diag]
                    )
                    plsc.store_scatter(out_v.at[buf], [diag, rowvec],
                                       val * scale)
                return c2

            lax.fori_loop(0, BL // LANES, sel_body, 0)
            pltpu.async_copy(out_v.at[buf], out_hbm.at[l, :, pl.ds(b0, BL)],
                             sem_o)

        issue(base, 0)

        def pair_body(p, carry):
            s0 = 2 * p
            issue(base + s0 + 1, 1)
            drain_g(0)
            proc(base + s0, 0, p, sem_o0)

            @pl.when(s0 + 2 < nblk)
            def _():
                issue(base + s0 + 2, 0)

            drain_g(1)
            proc(base + s0 + 1, 1, p, sem_o1)
            return carry

        lax.fori_loop(0, nblk // 2, pair_body, 0)

        l0, b00 = lb0(base + nblk - 2)
        pltpu.make_async_copy(
            out_v.at[0], out_hbm.at[l0, :, pl.ds(b00, BL)], sem_o0
        ).wait()
        l1, b01 = lb0(base + nblk - 1)
        pltpu.make_async_copy(
            out_v.at[1], out_hbm.at[l1, :, pl.ds(b01, BL)], sem_o1
        ).wait()

    return emb


def kernel(tokens, table):
    B, L = tokens.shape
    V, D = table.shape
    assert B % (K * SUB) == 0 and (L * B // (K * SUB)) % NW == 0
    assert SUB % D == 0 and TCH % LANES == 0
    assert (TCH * D) % SUB == 0

    tok_t = tokens.T.astype(jnp.int32)
    tbl_t = table.T
    scale = math.sqrt(D)
    vq = (V // SUB) * SUB
    tail16 = table[vq:].reshape(max((V - vq) * D // SUB, 1), SUB)
    tbl128 = _make_transpose_kernel(V, D)(tbl_t, tail16)
    out_t = _make_gather_kernel(L, B, D, scale)(tok_t, tbl128)
    return out_t.transpose(2, 0, 1)


# speedup vs baseline: 1.0827x; 1.0827x over previous
import functools
import math

import jax
import jax.numpy as jnp
from jax import lax
from jax.experimental import pallas as pl
from jax.experimental.pallas import tpu as pltpu
from jax.experimental.pallas import tpu_sc as plsc

NC = 2
NS = 16
NW = NC * NS
LANES = 16

SUB = 128
K = 2
TCH = 512


def _make_transpose_kernel(V, D):
    pack = SUB // D
    nch = V // TCH
    orows = TCH // pack
    niter = (nch + NW - 1) // NW
    niter2 = (niter + 1) // 2

    mesh = plsc.VectorSubcoreMesh(core_axis_name="c", subcore_axis_name="s")

    @functools.partial(
        pl.kernel,
        out_type=jax.ShapeDtypeStruct((V * D // SUB, SUB), jnp.float32),
        mesh=mesh,
        scratch_types=[
            pltpu.VMEM((2, D, TCH), jnp.float32),
            pltpu.VMEM((2, orows, SUB), jnp.float32),
            pltpu.SemaphoreType.DMA,
            pltpu.SemaphoreType.DMA,
            pltpu.SemaphoreType.DMA,
        ],
        compiler_params=pltpu.CompilerParams(needs_layout_passes=False),
    )
    def trans(tbl_t_hbm, tail_hbm, out_hbm, in_v, out_v, sem_i, so0, so1):
        wid = lax.axis_index("s") * NC + lax.axis_index("c")
        iota = lax.iota(jnp.int32, LANES)
        orow_off = iota >> 2
        ocol_base = (iota & (pack - 1)) * D

        def do_chunk(in_ref, out_ref):
            def gbody(g, c):
                vcol = iota + g * LANES
                orow = orow_off + g * (LANES // pack)
                for d0 in range(D):
                    diag = (iota + d0) & (D - 1)
                    val = plsc.load_gather(in_ref, [diag, vcol])
                    plsc.store_scatter(out_ref, [orow, ocol_base + diag], val)
                return c

            lax.fori_loop(0, TCH // LANES, gbody, 0)

        def issue(s, buf):
            ch = wid + s * NW

            @pl.when(ch < nch)
            def _():
                v0 = pl.multiple_of(ch * TCH, SUB)
                pltpu.async_copy(
                    tbl_t_hbm.at[:, pl.ds(v0, TCH)], in_v.at[buf], sem_i
                )

        def drain_i(s, buf):
            ch = wid + s * NW

            @pl.when(ch < nch)
            def _():
                v0 = pl.multiple_of(ch * TCH, SUB)
                pltpu.make_async_copy(
                    tbl_t_hbm.at[:, pl.ds(v0, TCH)], in_v.at[buf], sem_i
                ).wait()

        def proc(s, buf, p, sem_o):
            ch = wid + s * NW

            @pl.when(ch < nch)
            def _():
                @pl.when(p > 0)
                def _():
                    pltpu.make_async_copy(
                        out_v.at[buf], out_hbm.at[pl.ds(0, orows)], sem_o
                    ).wait()

                do_chunk(in_v.at[buf], out_v.at[buf])
                o0 = pl.multiple_of(ch * orows, orows)
                pltpu.async_copy(out_v.at[buf], out_hbm.at[pl.ds(o0, orows)],
                                 sem_o)

        issue(0, 0)

        def pair_body(p, carry):
            s0 = 2 * p
            issue(s0 + 1, 1)
            drain_i(s0, 0)
            proc(s0, 0, p, so0)
            issue(s0 + 2, 0)
            drain_i(s0 + 1, 1)
            proc(s0 + 1, 1, p, so1)
            return carry

        lax.fori_loop(0, niter2, pair_body, 0)

        assert nch >= 3 * NW
        pltpu.make_async_copy(
            out_v.at[0], out_hbm.at[pl.ds(0, orows)], so0
        ).wait()
        pltpu.make_async_copy(
            out_v.at[1], out_hbm.at[pl.ds(0, orows)], so1
        ).wait()

        rem = V - nch * TCH
        a = (rem // SUB) * SUB
        if a:
            @pl.when(wid == 0)
            def _():
                pltpu.sync_copy(
                    tbl_t_hbm.at[:, pl.ds(nch * TCH, a)],
                    in_v.at[0, :, pl.ds(0, a)],
                )
                def gbody(g, c):
                    vcol = iota + g * LANES
                    orow = orow_off + g * (LANES // pack)
                    for d0 in range(D):
                        diag = (iota + d0) & (D - 1)
                        val = plsc.load_gather(in_v.at[0], [diag, vcol])
                        plsc.store_scatter(out_v.at[0],
                                           [orow, ocol_base + diag], val)
                    return c

                lax.fori_loop(0, a // LANES, gbody, 0)
                pltpu.sync_copy(
                    out_v.at[0, pl.ds(0, a // pack)],
                    out_hbm.at[pl.ds((nch * TCH) // pack, a // pack)],
                )

        t16 = (V % SUB) * D // SUB
        if t16:
            @pl.when(wid == 1)
            def _():
                pltpu.sync_copy(tail_hbm, out_v.at[0, pl.ds(0, t16)])
                pltpu.sync_copy(
                    out_v.at[0, pl.ds(0, t16)],
                    out_hbm.at[pl.ds((V * D // SUB) - t16, t16)],
                )

    return trans


def _make_gather_kernel(L, B, D, scale):
    cpl = B // (K * SUB)
    nblk = L * cpl // NW
    assert nblk % 2 == 0
    pack = SUB // D
    BL = K * SUB

    mesh = plsc.VectorSubcoreMesh(core_axis_name="c", subcore_axis_name="s")

    @functools.partial(
        pl.kernel,
        out_type=jax.ShapeDtypeStruct((L, D, B), jnp.float32),
        mesh=mesh,
        scratch_types=[
            pltpu.VMEM((2, BL), jnp.int32),
            pltpu.VMEM((2, K, SUB), jnp.int32),
            pltpu.VMEM((2, BL // LANES, LANES), jnp.int32),
            pltpu.VMEM((2, BL, SUB), jnp.float32),
            pltpu.VMEM((2, D, BL), jnp.float32),
            pltpu.SemaphoreType.DMA,
            pltpu.SemaphoreType.DMA,
            pltpu.SemaphoreType.DMA,
        ],
        compiler_params=pltpu.CompilerParams(needs_layout_passes=False),
    )
    def emb(tokens_hbm, table_hbm, out_hbm, idv_v, idq_v, rem_v, rows_v,
            out_v, sem_g, sem_o0, sem_o1):
        wid = lax.axis_index("s") * NC + lax.axis_index("c")
        iota = lax.iota(jnp.int32, LANES)
        base = wid * nblk

        def lb0(bid):
            return bid // cpl, (bid % cpl) * BL

        def issue(bid, buf):
            l, b0 = lb0(bid)
            pltpu.sync_copy(tokens_hbm.at[l, pl.ds(b0, BL)], idv_v.at[buf])
            for j in range(K):
                for g in range(SUB // LANES):
                    iv = idv_v[buf, pl.ds(j * SUB + g * LANES, LANES)]
                    rem_v[buf, j * (SUB // LANES) + g, :] = iv & (pack - 1)
                    idq_v[buf, j, pl.ds(g * LANES, LANES)] = iv >> 2
            for j in range(K):
                pltpu.async_copy(
                    table_hbm.at[idq_v.at[buf, j]],
                    rows_v.at[buf, pl.ds(j * SUB, SUB)],
                    sem_g,
                )

        def drain_g(buf):
            for j in range(K):
                pltpu.make_async_copy(
                    table_hbm.at[idq_v.at[buf, j]],
                    rows_v.at[buf, pl.ds(j * SUB, SUB)],
                    sem_g,
                ).wait()

        def proc(bid, buf, p, sem_o):
            l, b0 = lb0(bid)

            @pl.when(p > 0)
            def _():
                l2, b02 = lb0(bid - 2)
                pltpu.make_async_copy(
                    out_v.at[buf], out_hbm.at[l2, :, pl.ds(b02, BL)], sem_o
                ).wait()

            def sel_body(jg, c2):
                rv = rem_v[buf, jg, :]
                colbase = rv * D
                rowvec = iota + jg * LANES
                for d0 in range(D):
                    diag = (iota + d0) & (D - 1)
                    val = plsc.load_gather(
                        rows_v.at[buf], [rowvec, colbase + diag]
                    )
                    plsc.store_scatter(out_v.at[buf], [diag, rowvec],
                                       val * scale)
                return c2

            lax.fori_loop(0, BL // LANES, sel_body, 0)
            pltpu.async_copy(out_v.at[buf], out_hbm.at[l, :, pl.ds(b0, BL)],
                             sem_o)

        issue(base, 0)

        def pair_body(p, carry):
            s0 = 2 * p
            issue(base + s0 + 1, 1)
            drain_g(0)
            proc(base + s0, 0, p, sem_o0)

            @pl.when(s0 + 2 < nblk)
            def _():
                issue(base + s0 + 2, 0)

            drain_g(1)
            proc(base + s0 + 1, 1, p, sem_o1)
            return carry

        lax.fori_loop(0, nblk // 2, pair_body, 0)

        l0, b00 = lb0(base + nblk - 2)
        pltpu.make_async_copy(
            out_v.at[0], out_hbm.at[l0, :, pl.ds(b00, BL)], sem_o0
        ).wait()
        l1, b01 = lb0(base + nblk - 1)
        pltpu.make_async_copy(
            out_v.at[1], out_hbm.at[l1, :, pl.ds(b01, BL)], sem_o1
        ).wait()

    return emb


def kernel(tokens, table):
    B, L = tokens.shape
    V, D = table.shape
    assert B % (K * SUB) == 0 and (L * B // (K * SUB)) % NW == 0
    assert SUB % D == 0 and TCH % LANES == 0
    assert (TCH * D) % SUB == 0

    tok_t = tokens.T.astype(jnp.int32)
    tbl_t = table.T
    scale = math.sqrt(D)
    vq = (V // SUB) * SUB
    tail16 = table[vq:].reshape(max((V - vq) * D // SUB, 1), SUB)
    tbl128 = _make_transpose_kernel(V, D)(tbl_t, tail16)
    out_t = _make_gather_kernel(L, B, D, scale)(tok_t, tbl128)
    return out_t.transpose(2, 0, 1)
